# baseline (device time: 361511 ns/iter reference)
import jax
import jax.numpy as jnp
from jax import lax
from jax.experimental import pallas as pl
from jax.experimental.pallas import tpu as pltpu

N_DEV = 8
_GELU_C = 0.7978845608028654


def kernel(x, w_mat):
    m_per, k = x.shape
    _, n_per = w_mat.shape

    def body(x_ref, w_ref, out_ref, gather_ref, send_sems, recv_sems):
        my = lax.axis_index("i")
        left = lax.rem(my - 1 + N_DEV, N_DEV)
        right = lax.rem(my + 1, N_DEV)

        barrier_sem = pltpu.get_barrier_semaphore()
        for nbr in (left, right):
            pl.semaphore_signal(
                barrier_sem, inc=1,
                device_id=(nbr,), device_id_type=pl.DeviceIdType.MESH,
            )
        pl.semaphore_wait(barrier_sem, 2)

        w_bf = w_ref[...].astype(jnp.bfloat16)

        def compute(origin):
            chunk = gather_ref[origin]
            y = jnp.dot(chunk, w_bf, preferred_element_type=jnp.float32)
            y = 0.5 * y * (1.0 + jnp.tanh(_GELU_C * (y + 0.044715 * y * y * y)))
            out_ref[pl.ds(origin * m_per, m_per), :] = y

        gather_ref[my] = x_ref[...].astype(jnp.bfloat16)
        compute(my)

        for h in range(N_DEV - 1):
            o_send = lax.rem(my - h + N_DEV, N_DEV)
            o_recv = lax.rem(my - h - 1 + N_DEV, N_DEV)
            rdma = pltpu.make_async_remote_copy(
                src_ref=gather_ref.at[o_send],
                dst_ref=gather_ref.at[o_send],
                send_sem=send_sems.at[h],
                recv_sem=recv_sems.at[h],
                device_id=(right,),
                device_id_type=pl.DeviceIdType.MESH,
            )
            rdma.start()
            rdma.wait()
            compute(o_recv)

    out_shape = jax.ShapeDtypeStruct((N_DEV * m_per, n_per), jnp.float32)
    return pl.pallas_call(
        body,
        out_shape=out_shape,
        in_specs=[
            pl.BlockSpec(memory_space=pltpu.VMEM),
            pl.BlockSpec(memory_space=pltpu.VMEM),
        ],
        out_specs=pl.BlockSpec(memory_space=pltpu.VMEM),
        scratch_shapes=[
            pltpu.VMEM((N_DEV, m_per, k), jnp.bfloat16),
            pltpu.SemaphoreType.DMA((N_DEV - 1,)),
            pltpu.SemaphoreType.DMA((N_DEV - 1,)),
        ],
        compiler_params=pltpu.CompilerParams(
            collective_id=0,
            vmem_limit_bytes=100 * 1024 * 1024,
        ),
    )(x, w_mat)


# device time: 189606 ns/iter; 1.9066x vs baseline; 1.9066x over previous
import jax
import jax.numpy as jnp
from jax import lax
from jax.experimental import pallas as pl
from jax.experimental.pallas import tpu as pltpu

N_DEV = 8
_GELU_C = 0.7978845608028654


def kernel(x, w_mat):
    m_per, k = x.shape
    _, n_per = w_mat.shape
    half = m_per // 2

    def body(x_ref, w_ref, out_ref, gather_ref,
             cw_send, cw_recv, ccw_send, ccw_recv):
        my = lax.axis_index("i")
        left = lax.rem(my - 1 + N_DEV, N_DEV)
        right = lax.rem(my + 1, N_DEV)

        barrier_sem = pltpu.get_barrier_semaphore()
        for nbr in (left, right):
            pl.semaphore_signal(
                barrier_sem, inc=1,
                device_id=(nbr,), device_id_type=pl.DeviceIdType.MESH,
            )
        pl.semaphore_wait(barrier_sem, 2)

        w_bf = w_ref[...].astype(jnp.bfloat16)

        def compute_half(d, origin):
            y = jnp.dot(gather_ref[d, origin], w_bf,
                        preferred_element_type=jnp.float32)
            y = 0.5 * y * (1.0 + jnp.tanh(_GELU_C * (y + 0.044715 * y * y * y)))
            out_ref[pl.ds(origin * m_per + d * half, half), :] = y

        gather_ref[0, my] = x_ref[0:half, :].astype(jnp.bfloat16)
        gather_ref[1, my] = x_ref[half:m_per, :].astype(jnp.bfloat16)

        for h in range(N_DEV - 1):
            o_cw = lax.rem(my - h + N_DEV, N_DEV)
            o_ccw = lax.rem(my + h, N_DEV)
            cw = pltpu.make_async_remote_copy(
                src_ref=gather_ref.at[0, o_cw],
                dst_ref=gather_ref.at[0, o_cw],
                send_sem=cw_send.at[h],
                recv_sem=cw_recv.at[h],
                device_id=(right,),
                device_id_type=pl.DeviceIdType.MESH,
            )
            ccw = pltpu.make_async_remote_copy(
                src_ref=gather_ref.at[1, o_ccw],
                dst_ref=gather_ref.at[1, o_ccw],
                send_sem=ccw_send.at[h],
                recv_sem=ccw_recv.at[h],
                device_id=(left,),
                device_id_type=pl.DeviceIdType.MESH,
            )
            cw.start()
            ccw.start()
            compute_half(0, o_cw)
            compute_half(1, o_ccw)
            cw.wait()
            ccw.wait()
        compute_half(0, lax.rem(my - (N_DEV - 1) + N_DEV, N_DEV))
        compute_half(1, lax.rem(my + (N_DEV - 1), N_DEV))

    out_shape = jax.ShapeDtypeStruct((N_DEV * m_per, n_per), jnp.float32)
    return pl.pallas_call(
        body,
        out_shape=out_shape,
        in_specs=[
            pl.BlockSpec(memory_space=pltpu.VMEM),
            pl.BlockSpec(memory_space=pltpu.VMEM),
        ],
        out_specs=pl.BlockSpec(memory_space=pltpu.VMEM),
        scratch_shapes=[
            pltpu.VMEM((2, N_DEV, half, k), jnp.bfloat16),
            pltpu.SemaphoreType.DMA((N_DEV - 1,)),
            pltpu.SemaphoreType.DMA((N_DEV - 1,)),
            pltpu.SemaphoreType.DMA((N_DEV - 1,)),
            pltpu.SemaphoreType.DMA((N_DEV - 1,)),
        ],
        compiler_params=pltpu.CompilerParams(
            collective_id=0,
            vmem_limit_bytes=100 * 1024 * 1024,
        ),
    )(x, w_mat)


# device time: 189425 ns/iter; 1.9085x vs baseline; 1.0010x over previous
import jax
import jax.numpy as jnp
from jax import lax
from jax.experimental import pallas as pl
from jax.experimental.pallas import tpu as pltpu

N_DEV = 8
_GELU_C = 0.7978845608028654


def kernel(x, w_mat):
    m_per, k = x.shape
    _, n_per = w_mat.shape
    half = m_per // 2

    def body(x_ref, w_ref, out_ref, gather_ref,
             cw_send, cw_recv, ccw_send, ccw_recv):
        my = lax.axis_index("i")
        left = lax.rem(my - 1 + N_DEV, N_DEV)
        right = lax.rem(my + 1, N_DEV)

        barrier_sem = pltpu.get_barrier_semaphore()
        for nbr in (left, right):
            pl.semaphore_signal(
                barrier_sem, inc=1,
                device_id=(nbr,), device_id_type=pl.DeviceIdType.MESH,
            )
        pl.semaphore_wait(barrier_sem, 2)

        def make_rdma(h):
            o_cw = lax.rem(my - h + N_DEV, N_DEV)
            o_ccw = lax.rem(my + h, N_DEV)
            cw = pltpu.make_async_remote_copy(
                src_ref=gather_ref.at[0, o_cw],
                dst_ref=gather_ref.at[0, o_cw],
                send_sem=cw_send.at[h],
                recv_sem=cw_recv.at[h],
                device_id=(right,),
                device_id_type=pl.DeviceIdType.MESH,
            )
            ccw = pltpu.make_async_remote_copy(
                src_ref=gather_ref.at[1, o_ccw],
                dst_ref=gather_ref.at[1, o_ccw],
                send_sem=ccw_send.at[h],
                recv_sem=ccw_recv.at[h],
                device_id=(left,),
                device_id_type=pl.DeviceIdType.MESH,
            )
            return o_cw, o_ccw, cw, ccw

        gather_ref[0, my] = x_ref[0:half, :].astype(jnp.bfloat16)
        gather_ref[1, my] = x_ref[half:m_per, :].astype(jnp.bfloat16)

        _, _, cw0, ccw0 = make_rdma(0)
        cw0.start()
        ccw0.start()

        w_bf = w_ref[...].astype(jnp.bfloat16)

        def compute_half(d, origin):
            y = jnp.dot(gather_ref[d, origin], w_bf,
                        preferred_element_type=jnp.float32)
            y = 0.5 * y * (1.0 + jnp.tanh(_GELU_C * (y + 0.044715 * y * y * y)))
            out_ref[pl.ds(origin * m_per + d * half, half), :] = y

        rdmas = [(cw0, ccw0)]
        for h in range(N_DEV - 1):
            o_cw = lax.rem(my - h + N_DEV, N_DEV)
            o_ccw = lax.rem(my + h, N_DEV)
            compute_half(0, o_cw)
            compute_half(1, o_ccw)
            cw, ccw = rdmas[h]
            cw.wait_recv()
            ccw.wait_recv()
            if h + 1 < N_DEV - 1:
                _, _, cw_n, ccw_n = make_rdma(h + 1)
                cw_n.start()
                ccw_n.start()
                rdmas.append((cw_n, ccw_n))
        compute_half(0, lax.rem(my - (N_DEV - 1) + N_DEV, N_DEV))
        compute_half(1, lax.rem(my + (N_DEV - 1), N_DEV))
        for cw, ccw in rdmas:
            cw.wait_send()
            ccw.wait_send()

    out_shape = jax.ShapeDtypeStruct((N_DEV * m_per, n_per), jnp.float32)
    return pl.pallas_call(
        body,
        out_shape=out_shape,
        in_specs=[
            pl.BlockSpec(memory_space=pltpu.VMEM),
            pl.BlockSpec(memory_space=pltpu.VMEM),
        ],
        out_specs=pl.BlockSpec(memory_space=pltpu.VMEM),
        scratch_shapes=[
            pltpu.VMEM((2, N_DEV, half, k), jnp.bfloat16),
            pltpu.SemaphoreType.DMA((N_DEV - 1,)),
            pltpu.SemaphoreType.DMA((N_DEV - 1,)),
            pltpu.SemaphoreType.DMA((N_DEV - 1,)),
            pltpu.SemaphoreType.DMA((N_DEV - 1,)),
        ],
        compiler_params=pltpu.CompilerParams(
            collective_id=0,
            vmem_limit_bytes=100 * 1024 * 1024,
        ),
    )(x, w_mat)


# device time: 176737 ns/iter; 2.0455x vs baseline; 1.0718x over previous
import jax
import jax.numpy as jnp
from jax import lax
from jax.experimental import pallas as pl
from jax.experimental.pallas import tpu as pltpu

N_DEV = 8
NPART = 2
_GELU_C = 0.7978845608028654


def kernel(x, w_mat):
    m_per, k = x.shape
    _, n_per = w_mat.shape
    half = m_per // 2
    part = half // NPART

    def body(x_ref, w_ref, out_ref, gather_ref,
             cw_send, cw_recv, ccw_send, ccw_recv):
        my = lax.axis_index("i")
        left = lax.rem(my - 1 + N_DEV, N_DEV)
        right = lax.rem(my + 1, N_DEV)

        barrier_sem = pltpu.get_barrier_semaphore()
        for nbr in (left, right):
            pl.semaphore_signal(
                barrier_sem, inc=1,
                device_id=(nbr,), device_id_type=pl.DeviceIdType.MESH,
            )
        pl.semaphore_wait(barrier_sem, 2)

        def o_of(d, h):
            return lax.rem(my - h + N_DEV, N_DEV) if d == 0 else \
                lax.rem(my + h, N_DEV)

        def make_rdma(d, h, j):
            o = o_of(d, h)
            send = (cw_send, ccw_send)[d]
            recv = (cw_recv, ccw_recv)[d]
            return pltpu.make_async_remote_copy(
                src_ref=gather_ref.at[d, o, j],
                dst_ref=gather_ref.at[d, o, j],
                send_sem=send.at[h, j],
                recv_sem=recv.at[h, j],
                device_id=(right if d == 0 else left,),
                device_id_type=pl.DeviceIdType.MESH,
            )

        for d in range(2):
            for j in range(NPART):
                r0 = d * half + j * part
                gather_ref[d, my, j] = x_ref[r0:r0 + part, :].astype(jnp.bfloat16)

        rdmas = {}
        for j in range(NPART):
            for d in range(2):
                rdmas[(d, 0, j)] = make_rdma(d, 0, j)
                rdmas[(d, 0, j)].start()

        w_bf = w_ref[...].astype(jnp.bfloat16)

        def compute_half(d, origin):
            chunk = gather_ref[d, origin].reshape(half, k)
            y = jnp.dot(chunk, w_bf, preferred_element_type=jnp.float32)
            y = 0.5 * y * (1.0 + jnp.tanh(_GELU_C * (y + 0.044715 * y * y * y)))
            out_ref[pl.ds(origin * m_per + d * half, half), :] = y

        for h in range(N_DEV - 1):
            compute_half(0, o_of(0, h))
            compute_half(1, o_of(1, h))
            for j in range(NPART):
                for d in range(2):
                    rdmas[(d, h, j)].wait_recv()
                    if h + 1 < N_DEV - 1:
                        r = make_rdma(d, h + 1, j)
                        rdmas[(d, h + 1, j)] = r
                        r.start()
        compute_half(0, o_of(0, N_DEV - 1))
        compute_half(1, o_of(1, N_DEV - 1))
        for r in rdmas.values():
            r.wait_send()

    out_shape = jax.ShapeDtypeStruct((N_DEV * m_per, n_per), jnp.float32)
    return pl.pallas_call(
        body,
        out_shape=out_shape,
        in_specs=[
            pl.BlockSpec(memory_space=pltpu.VMEM),
            pl.BlockSpec(memory_space=pltpu.VMEM),
        ],
        out_specs=pl.BlockSpec(memory_space=pltpu.VMEM),
        scratch_shapes=[
            pltpu.VMEM((2, N_DEV, NPART, part, k), jnp.bfloat16),
            pltpu.SemaphoreType.DMA((N_DEV - 1, NPART)),
            pltpu.SemaphoreType.DMA((N_DEV - 1, NPART)),
            pltpu.SemaphoreType.DMA((N_DEV - 1, NPART)),
            pltpu.SemaphoreType.DMA((N_DEV - 1, NPART)),
        ],
        compiler_params=pltpu.CompilerParams(
            collective_id=0,
            vmem_limit_bytes=100 * 1024 * 1024,
        ),
    )(x, w_mat)
